# baseline (speedup 1.0000x reference)
import functools

import jax
import jax.numpy as jnp
from jax import lax
from jax.experimental import pallas as pl
from jax.experimental.pallas import tpu as pltpu, tpu_sc as plsc

_Q = 16384
_K = 256
_D = 128
_NST = 2
_TH = 2048
_GRID = _Q // (_NST * _TH)

_CHUNK = 128


def _half(s, q, t):
    scores = jax.lax.dot_general(
        s, q, (((1,), (1,)), ((), ())), preferred_element_type=jnp.float32
    )
    iota = jax.lax.broadcasted_iota(jnp.int32, scores.shape, 0)
    m = jnp.max(scores, axis=0, keepdims=True)
    preds = jnp.min(jnp.where(scores == m, iota, _K), axis=0, keepdims=True)
    correct = preds == t
    lse = jnp.log(jnp.sum(jnp.exp(scores), axis=0, keepdims=True))
    return correct, jnp.sum(lse)


def _head_kernel(*refs):
    q_refs = refs[:_NST]
    s_ref = refs[_NST]
    t_refs = refs[_NST + 1:2 * _NST + 1]
    c_ref = refs[2 * _NST + 1]
    lse_ref = refs[2 * _NST + 2]
    i = pl.program_id(0)
    s = s_ref[...]
    lse_sum = None
    for j, (q_ref, t_ref) in enumerate(zip(q_refs, t_refs)):
        c, n = _half(s, q_ref[...], t_ref[0, :, :])
        c_ref[j, :, :] = c
        lse_sum = n if lse_sum is None else lse_sum + n
    prev = jnp.where(i == 0, jnp.zeros_like(lse_ref[...]), lse_ref[...])
    lse_ref[...] = prev + lse_sum


def _qspec(j):
    return pl.BlockSpec((_TH, _D), lambda i, j=j: (_NST * i + j, 0))


def _tspec(j):
    return pl.BlockSpec((1, 1, _TH), lambda i, j=j: (_NST * i + j, 0, 0))


_info = plsc.get_sparse_core_info()
_NC = _info.num_cores
_NS = _info.num_subcores
_NW = _NC * _NS
_BPW = _Q // _NW


def _make_segsum():
    mesh = plsc.VectorSubcoreMesh(core_axis_name="c", subcore_axis_name="s")

    @functools.partial(
        pl.kernel, mesh=mesh,
        out_type=jax.ShapeDtypeStruct((_NC, _K, _D), jnp.float32),
        scratch_types=[
            pltpu.VMEM((_CHUNK,), jnp.int32),
            pltpu.VMEM((_CHUNK, _D), jnp.float32),
            pltpu.VMEM_SHARED((_K, _D), jnp.float32),
        ],
    )
    def segsum(q_hbm, t_hbm, zero_hbm, out_hbm, idx_v, rows_v, agg_s):
        cid = lax.axis_index("c")
        sid = lax.axis_index("s")
        wid = sid * _NC + cid

        @pl.when(sid == 0)
        def _():
            pltpu.sync_copy(zero_hbm, agg_s)

        plsc.subcore_barrier()
        for ch in range(_BPW // _CHUNK):
            base = wid * _BPW + ch * _CHUNK
            pltpu.sync_copy(t_hbm.at[pl.ds(base, _CHUNK)], idx_v)
            pltpu.sync_copy(q_hbm.at[pl.ds(base, _CHUNK)], rows_v)
            pltpu.sync_copy(rows_v, agg_s.at[idx_v], add=True)
        plsc.subcore_barrier()

        @pl.when(sid == 0)
        def _():
            pltpu.sync_copy(agg_s, out_hbm.at[cid])

    return segsum


_segsum = _make_segsum()


@jax.jit
def kernel(query_reps, support_reps, target_ids):
    t32 = target_ids.astype(jnp.int32)
    targets = t32.reshape(_NST * _GRID, 1, _TH)
    agg2 = _segsum(query_reps, t32, jnp.zeros((_K, _D), jnp.float32))
    correct, lse_sum = pl.pallas_call(
        _head_kernel,
        grid=(_GRID,),
        in_specs=(
            [_qspec(j) for j in range(_NST)]
            + [pl.BlockSpec((_K, _D), lambda i: (0, 0))]
            + [_tspec(j) for j in range(_NST)]
        ),
        out_specs=[
            pl.BlockSpec((_NST, 1, _TH), lambda i: (i, 0, 0)),
            pl.BlockSpec((1, 128), lambda i: (0, 0)),
        ],
        out_shape=[
            jax.ShapeDtypeStruct((_NST * _GRID, 1, _TH), jnp.bool_),
            jax.ShapeDtypeStruct((1, 128), jnp.float32),
        ],
    )(*([query_reps] * _NST), support_reps, *([targets] * _NST))
    tgt_total = jnp.sum((agg2[0] + agg2[1]) * support_reps)
    loss = (lse_sum[0, 0] - tgt_total) / _Q
    return (loss, correct.reshape(_Q))


# speedup vs baseline: 2.5267x; 2.5267x over previous
import jax
import jax.numpy as jnp
from jax.experimental import pallas as pl

_Q = 16384
_K = 256
_D = 128
_NST = 2
_TH = 2048
_GRID = _Q // (_NST * _TH)


def _half(s, q, t):
    scores = jax.lax.dot_general(
        s, q, (((1,), (1,)), ((), ())), preferred_element_type=jnp.float32
    )
    iota = jax.lax.broadcasted_iota(jnp.int32, scores.shape, 0)
    m = jnp.max(scores, axis=0, keepdims=True)
    tgt = jnp.sum(jnp.where(iota == t, scores, 0.0), axis=0, keepdims=True)
    preds = jnp.min(jnp.where(scores == m, iota, _K), axis=0, keepdims=True)
    correct = preds == t
    lse = jnp.log(jnp.sum(jnp.exp(scores), axis=0, keepdims=True))
    return correct, jnp.sum(lse - tgt)


def _head_kernel(*refs):
    q_refs = refs[:_NST]
    s_ref = refs[_NST]
    t_refs = refs[_NST + 1:2 * _NST + 1]
    c_ref = refs[2 * _NST + 1]
    loss_ref = refs[2 * _NST + 2]
    i = pl.program_id(0)
    g = pl.num_programs(0)
    s = s_ref[...]
    nll = None
    for j, (q_ref, t_ref) in enumerate(zip(q_refs, t_refs)):
        c, n = _half(s, q_ref[...], t_ref[0, :, :])
        c_ref[j, :, :] = c
        nll = n if nll is None else nll + n
    prev = jnp.where(i == 0, jnp.zeros_like(loss_ref[...]), loss_ref[...])
    acc = prev + nll
    loss_ref[...] = jnp.where(i == g - 1, acc / _Q, acc)


def _qspec(j):
    return pl.BlockSpec((_TH, _D), lambda i, j=j: (_NST * i + j, 0))


def _tspec(j):
    return pl.BlockSpec((1, 1, _TH), lambda i, j=j: (_NST * i + j, 0, 0))


@jax.jit
def kernel(query_reps, support_reps, target_ids):
    targets = target_ids.astype(jnp.int32).reshape(_NST * _GRID, 1, _TH)
    correct, loss = pl.pallas_call(
        _head_kernel,
        grid=(_GRID,),
        in_specs=(
            [_qspec(j) for j in range(_NST)]
            + [pl.BlockSpec((_K, _D), lambda i: (0, 0))]
            + [_tspec(j) for j in range(_NST)]
        ),
        out_specs=[
            pl.BlockSpec((_NST, 1, _TH), lambda i: (i, 0, 0)),
            pl.BlockSpec((1, 128), lambda i: (0, 0)),
        ],
        out_shape=[
            jax.ShapeDtypeStruct((_NST * _GRID, 1, _TH), jnp.bool_),
            jax.ShapeDtypeStruct((1, 128), jnp.float32),
        ],
    )(*([query_reps] * _NST), support_reps, *([targets] * _NST))
    return (loss[0, 0], correct.reshape(_Q))
